# baseline (speedup 1.0000x reference)
import functools

import jax
import jax.numpy as jnp
from jax.experimental import pallas as pl

BS, T, N, K = 16, 64, 1024, 32
NEG = -1e30


def _sttn_kernel(x_ref, w_ref, b_ref, out_ref):
    xs = x_ref[0]
    mean = jnp.mean(xs, axis=0, keepdims=True)
    xc = xs - mean
    nrm = jnp.sqrt(jnp.sum(xc * xc, axis=0, keepdims=True)) + 1e-6
    xn = xc / nrm
    adj = jax.lax.dot_general(
        xn, xn, (((0,), (0,)), ((), ())),
        preferred_element_type=jnp.float32)

    kf = jnp.float32(K)

    def body(_, c):
        lo, hi = c
        mid = 0.5 * (lo + hi)
        cnt = jnp.sum(jnp.where(adj >= mid, 1.0, 0.0), axis=0,
                      keepdims=True)
        ge = cnt >= kf
        return jnp.where(ge, mid, lo), jnp.where(ge, hi, mid)

    BISECT = 15
    thresh, _ = jax.lax.fori_loop(
        0, BISECT, body,
        (jnp.full((1, N), -1.0, jnp.float32),
         jnp.full((1, N), 1.0, jnp.float32)),
        unroll=True)

    p = jnp.where(adj >= thresh, jnp.exp(adj - 1.0), 0.0)

    xs1 = jnp.concatenate(
        [xs, jnp.ones((1, N), jnp.float32)], axis=0)
    agg_raw = jax.lax.dot_general(
        xs1, p, (((1,), (0,)), ((), ())),
        preferred_element_type=jnp.float32)
    aggT = agg_raw[:T] / agg_raw[T:T + 1]

    w = w_ref[...]
    w1 = w[:, :T]
    w2 = w[:, T:]
    outT = (
        jax.lax.dot_general(w1, xs, (((1,), (0,)), ((), ())),
                            preferred_element_type=jnp.float32)
        + jax.lax.dot_general(w2, aggT, (((1,), (0,)), ((), ())),
                              preferred_element_type=jnp.float32)
        + b_ref[...].reshape(T, 1)
    )
    out_ref[0] = outT


@jax.jit
def kernel(x_pr, W, b):
    x_tn = x_pr.reshape(BS, T, N)
    out = pl.pallas_call(
        _sttn_kernel,
        grid=(BS,),
        in_specs=[
            pl.BlockSpec((1, T, N), lambda i: (i, 0, 0)),
            pl.BlockSpec((T, 2 * T), lambda i: (0, 0)),
            pl.BlockSpec((1, T), lambda i: (0, 0)),
        ],
        out_specs=pl.BlockSpec((1, T, N), lambda i: (i, 0, 0)),
        out_shape=jax.ShapeDtypeStruct((BS, T, N), jnp.float32),
    )(x_tn, W, b.reshape(1, T))
    return out


# speedup vs baseline: 1.1531x; 1.1531x over previous
import functools

import jax
import jax.numpy as jnp
from jax.experimental import pallas as pl

BS, T, N, K = 16, 64, 1024, 32
NEG = -1e30


def _sttn_kernel(x_ref, w_ref, b_ref, out_ref):
    xs = x_ref[0]
    mean = jnp.mean(xs, axis=0, keepdims=True)
    xc = xs - mean
    nrm = jnp.sqrt(jnp.sum(xc * xc, axis=0, keepdims=True)) + 1e-6
    xn = xc / nrm
    adj = jax.lax.dot_general(
        xn, xn, (((0,), (0,)), ((), ())),
        preferred_element_type=jnp.float32)

    kf = jnp.float32(K)

    def body(_, c):
        lo, hi = c
        mid = 0.5 * (lo + hi)
        cnt = jnp.sum(jnp.where(adj >= mid, 1.0, 0.0), axis=1,
                      keepdims=True)
        ge = cnt >= kf
        return jnp.where(ge, mid, lo), jnp.where(ge, hi, mid)

    BISECT = 15
    thresh, _ = jax.lax.fori_loop(
        0, BISECT, body,
        (jnp.full((N, 1), -1.0, jnp.float32),
         jnp.full((N, 1), 1.0, jnp.float32)),
        unroll=True)

    p = jnp.where(adj >= thresh, jnp.exp(adj - 1.0), 0.0)

    xs1 = jnp.concatenate(
        [xs, jnp.ones((1, N), jnp.float32)], axis=0)
    agg_raw = jax.lax.dot_general(
        xs1, p, (((1,), (1,)), ((), ())),
        preferred_element_type=jnp.float32)
    aggT = agg_raw[:T] / agg_raw[T:T + 1]

    w = w_ref[...]
    w1 = w[:, :T]
    w2 = w[:, T:]
    outT = (
        jax.lax.dot_general(w1, xs, (((1,), (0,)), ((), ())),
                            preferred_element_type=jnp.float32)
        + jax.lax.dot_general(w2, aggT, (((1,), (0,)), ((), ())),
                              preferred_element_type=jnp.float32)
        + b_ref[...].reshape(T, 1)
    )
    out_ref[0] = outT


@jax.jit
def kernel(x_pr, W, b):
    x_tn = x_pr.reshape(BS, T, N)
    out = pl.pallas_call(
        _sttn_kernel,
        grid=(BS,),
        in_specs=[
            pl.BlockSpec((1, T, N), lambda i: (i, 0, 0)),
            pl.BlockSpec((T, 2 * T), lambda i: (0, 0)),
            pl.BlockSpec((1, T), lambda i: (0, 0)),
        ],
        out_specs=pl.BlockSpec((1, T, N), lambda i: (i, 0, 0)),
        out_shape=jax.ShapeDtypeStruct((BS, T, N), jnp.float32),
    )(x_tn, W, b.reshape(1, T))
    return out
